# baseline (speedup 1.0000x reference)
import functools

import jax
import jax.numpy as jnp
from jax import lax
from jax.experimental import pallas as pl
from jax.experimental.pallas import tpu as pltpu
from jax.experimental.pallas import tpu_sc as plsc

N = 524288
D = 3
T = 64
RES = 128
B = 64
RADIUS = 1.0

NW = 32
L = 16
PPW = N // NW
HSZ = RES * T
GSZ = 32
NGROUPS = PPW // GSZ
NTG = T // L

SLOTS = NW * B
CH = 128
NSTEPS = SLOTS // CH


def _sc_hist(x0, x1, x2, batch, v_flat):
    mesh = plsc.VectorSubcoreMesh(
        core_axis_name="c", subcore_axis_name="s", num_cores=2
    )

    @functools.partial(
        pl.kernel,
        mesh=mesh,
        compiler_params=pltpu.CompilerParams(needs_layout_passes=False),
        out_type=(
            jax.ShapeDtypeStruct((NW, B, HSZ), jnp.float32),
            jax.ShapeDtypeStruct((NW, B), jnp.int32),
        ),
        scratch_types=[
            pltpu.VMEM((PPW,), jnp.float32),
            pltpu.VMEM((PPW,), jnp.float32),
            pltpu.VMEM((PPW,), jnp.float32),
            pltpu.VMEM((PPW,), jnp.int32),
            pltpu.VMEM((D * T,), jnp.float32),
            pltpu.VMEM((HSZ,), jnp.float32),
            pltpu.VMEM((B,), jnp.int32),
        ],
    )
    def k(x0_hbm, x1_hbm, x2_hbm, b_hbm, v_hbm, part_hbm, ids_hbm,
          xb0, xb1, xb2, bb, vb, hist, idsb):
        cid = lax.axis_index("c")
        sid = lax.axis_index("s")
        wid = sid * 2 + cid
        start = wid * PPW

        pltpu.sync_copy(x0_hbm.at[pl.ds(start, PPW)], xb0)
        pltpu.sync_copy(x1_hbm.at[pl.ds(start, PPW)], xb1)
        pltpu.sync_copy(x2_hbm.at[pl.ds(start, PPW)], xb2)
        pltpu.sync_copy(b_hbm.at[pl.ds(start, PPW)], bb)
        pltpu.sync_copy(v_hbm, vb)

        zeros16 = jnp.zeros((L,), jnp.float32)
        ones16 = jnp.ones((L,), jnp.float32)
        neg16 = jnp.full((L,), -1, jnp.int32)
        lane = lax.iota(jnp.int32, L)
        lane0 = lane == 0

        c64 = jnp.full((L,), jnp.float32(T))
        vvec = [
            [vb[pl.ds(d * T + g * L, L)] * jnp.float32(T) for d in range(D)]
            for g in range(NTG)
        ]
        tvec = [lane + g * L for g in range(NTG)]

        def zero_hist():
            def zb(i, c):
                hist[pl.ds(i * L, L)] = zeros16
                return c
            lax.fori_loop(0, HSZ // L, zb, 0, unroll=8)

        def flush(cur_b, slot):
            pltpu.sync_copy(hist, part_hbm.at[wid, slot])
            plsc.store_scatter(
                idsb,
                [jnp.full((L,), slot, jnp.int32)],
                jnp.full((L,), cur_b, jnp.int32),
                mask=lane0,
            )
            zero_hist()

        def load_block(base):
            return [
                [xb[pl.ds(base + h * L, L)] for h in range(GSZ // L)]
                for xb in (xb0, xb1, xb2)
            ]

        def process(coords, kk):
            xs, ys, zs = coords
            h, j = kk // L, kk % L
            x0 = xs[h][j]
            x1 = ys[h][j]
            x2 = zs[h][j]
            sis = []
            for g in range(NTG):
                v0, v1, v2 = vvec[g]
                y = x0 * v0 + (x1 * v1 + (x2 * v2 + c64))
                yc = jnp.minimum(jnp.maximum(y, 0.0), jnp.float32(RES - 1))
                yi = yc.astype(jnp.int32)
                sis.append(yi * T + tvec[g])
            for si in sis:
                plsc.addupdate_scatter(hist, [si], ones16)

        for j in range(B // L):
            idsb[pl.ds(j * L, L)] = neg16
        zero_hist()

        def group_body(g, carry):
            base = g * GSZ
            bvecs = [bb[pl.ds(base + h * L, L)] for h in range(GSZ // L)]
            bf = bvecs[0][0]
            bl = bvecs[-1][L - 1]
            cur_b = carry[0]
            same = jnp.logical_and(bf == cur_b, bl == cur_b)

            def fast(c):
                coords = load_block(base)
                for kk in range(GSZ):
                    process(coords, kk)
                return c

            def slow(c):
                cb, sl = c
                coords = load_block(base)
                for kk in range(GSZ):
                    bp = bvecs[kk // L][kk % L]

                    def do_flush(cs):
                        flush(cs[0], cs[1])
                        return (bp, cs[1] + 1)

                    cb, sl = lax.cond(bp != cb, do_flush, lambda cs: cs, (cb, sl))
                    process(coords, kk)
                return (cb, sl)

            return lax.cond(same, fast, slow, carry)

        b0 = bb[pl.ds(0, L)][0]
        cur_b, slot = lax.fori_loop(
            0, NGROUPS, group_body, (b0, jnp.int32(0))
        )
        flush(cur_b, slot)
        pltpu.sync_copy(idsb, ids_hbm.at[wid])

    return k(x0, x1, x2, batch, v_flat)


def _tc_reduce(parts, ids3, idcol):
    def body(ids_ref, idcol_ref, parts_ref, out_ref, acc_ref):
        kk = pl.program_id(0)
        ids_row = ids_ref[0]
        idc = idcol_ref[...]
        rows = jnp.where(idc[:, 0:1] >= 0, parts_ref[...], 0.0)
        bio = lax.broadcasted_iota(jnp.int32, (B, CH), 0)
        mask = (jnp.broadcast_to(ids_row, (B, CH)) == bio).astype(jnp.float32)
        contrib = jax.lax.dot(
            mask, rows,
            precision=lax.Precision.HIGHEST,
            preferred_element_type=jnp.float32,
        )

        @pl.when(kk == 0)
        def _():
            acc_ref[...] = contrib

        @pl.when(kk > 0)
        def _():
            acc_ref[...] = acc_ref[...] + contrib

        @pl.when(kk == NSTEPS - 1)
        def _():
            h = acc_ref[...]
            s = 1
            while s < RES:
                h = h + jnp.concatenate(
                    [jnp.zeros((B, s * T), jnp.float32), h[:, : (RES - s) * T]],
                    axis=1,
                )
                s *= 2
            out_ref[...] = h

    return pl.pallas_call(
        body,
        grid=(NSTEPS,),
        in_specs=[
            pl.BlockSpec((1, 1, CH), lambda k: (k, 0, 0)),
            pl.BlockSpec((CH, 128), lambda k: (k, 0)),
            pl.BlockSpec((CH, HSZ), lambda k: (k, 0)),
        ],
        out_specs=pl.BlockSpec((B, HSZ), lambda k: (0, 0)),
        out_shape=jax.ShapeDtypeStruct((B, HSZ), jnp.float32),
        scratch_shapes=[pltpu.VMEM((B, HSZ), jnp.float32)],
    )(ids3, idcol, parts)


def kernel(x, batch, v):
    xt = x.T
    b32 = batch.astype(jnp.int32)
    vf = v.reshape(-1)
    parts, ids = _sc_hist(xt[0], xt[1], xt[2], b32, vf)
    ids_flat = ids.reshape(SLOTS)
    flat = _tc_reduce(
        parts.reshape(SLOTS, HSZ),
        ids_flat.reshape(NSTEPS, 1, CH),
        jnp.broadcast_to(ids_flat[:, None], (SLOTS, 128)),
    )
    return flat.reshape(B, RES, T)


# speedup vs baseline: 1.3881x; 1.3881x over previous
import functools

import jax
import jax.numpy as jnp
from jax import lax
from jax.experimental import pallas as pl
from jax.experimental.pallas import tpu as pltpu
from jax.experimental.pallas import tpu_sc as plsc

N = 524288
D = 3
T = 64
RES = 128
B = 64
RADIUS = 1.0

NW = 32
L = 16
PPW = N // NW
HSZ = RES * T
GSZ = 16
NGROUPS = PPW // GSZ
NTG = T // L

SLOTS = NW * B
CH = 128
NSTEPS = SLOTS // CH


def _sc_hist(x0, x1, x2, batch, v_flat):
    mesh = plsc.VectorSubcoreMesh(
        core_axis_name="c", subcore_axis_name="s", num_cores=2
    )

    @functools.partial(
        pl.kernel,
        mesh=mesh,
        compiler_params=pltpu.CompilerParams(needs_layout_passes=False),
        out_type=(
            jax.ShapeDtypeStruct((NW, B, HSZ), jnp.float32),
            jax.ShapeDtypeStruct((NW, B), jnp.int32),
        ),
        scratch_types=[
            pltpu.VMEM((PPW,), jnp.float32),
            pltpu.VMEM((PPW,), jnp.float32),
            pltpu.VMEM((PPW,), jnp.float32),
            pltpu.VMEM((PPW,), jnp.int32),
            pltpu.VMEM((D * T,), jnp.float32),
            pltpu.VMEM((HSZ,), jnp.float32),
            pltpu.VMEM((B,), jnp.int32),
        ],
    )
    def k(x0_hbm, x1_hbm, x2_hbm, b_hbm, v_hbm, part_hbm, ids_hbm,
          xb0, xb1, xb2, bb, vb, hist, idsb):
        cid = lax.axis_index("c")
        sid = lax.axis_index("s")
        wid = sid * 2 + cid
        start = wid * PPW

        pltpu.sync_copy(x0_hbm.at[pl.ds(start, PPW)], xb0)
        pltpu.sync_copy(x1_hbm.at[pl.ds(start, PPW)], xb1)
        pltpu.sync_copy(x2_hbm.at[pl.ds(start, PPW)], xb2)
        pltpu.sync_copy(b_hbm.at[pl.ds(start, PPW)], bb)
        pltpu.sync_copy(v_hbm, vb)

        zeros16 = jnp.zeros((L,), jnp.float32)
        ones16 = jnp.ones((L,), jnp.float32)
        neg16 = jnp.full((L,), -1, jnp.int32)
        lane = lax.iota(jnp.int32, L)
        lane0 = lane == 0

        c64 = jnp.full((L,), jnp.float32(T))
        vvec = [
            [vb[pl.ds(d * T + g * L, L)] * jnp.float32(T) for d in range(D)]
            for g in range(NTG)
        ]
        tvec = [lane + g * L for g in range(NTG)]

        def zero_hist():
            def zb(i, c):
                hist[pl.ds(i * L, L)] = zeros16
                return c
            lax.fori_loop(0, HSZ // L, zb, 0, unroll=8)

        def flush(cur_b, slot):
            pltpu.sync_copy(hist, part_hbm.at[wid, slot])
            plsc.store_scatter(
                idsb,
                [jnp.full((L,), slot, jnp.int32)],
                jnp.full((L,), cur_b, jnp.int32),
                mask=lane0,
            )
            zero_hist()

        def load_block(base):
            xa = xb0[pl.ds(base, L)]
            ya = xb1[pl.ds(base, L)]
            za = xb2[pl.ds(base, L)]
            return xa, ya, za

        def process(coords, kk):
            xa, ya, za = coords
            x0 = xa[kk]
            x1 = ya[kk]
            x2 = za[kk]
            sis = []
            for g in range(NTG):
                v0, v1, v2 = vvec[g]
                y = x0 * v0 + (x1 * v1 + (x2 * v2 + c64))
                yc = jnp.minimum(jnp.maximum(y, 0.0), jnp.float32(RES - 1))
                yi = yc.astype(jnp.int32)
                sis.append(yi * T + tvec[g])
            for si in sis:
                plsc.addupdate_scatter(hist, [si], ones16)

        for j in range(B // L):
            idsb[pl.ds(j * L, L)] = neg16
        zero_hist()

        def group_body(g, carry):
            base = g * GSZ
            bvec = bb[pl.ds(base, GSZ)]
            bf = bvec[0]
            bl = bvec[GSZ - 1]
            cur_b = carry[0]
            same = jnp.logical_and(bf == cur_b, bl == cur_b)

            def fast(c):
                coords = load_block(base)
                for kk in range(GSZ):
                    process(coords, kk)
                return c

            def slow(c):
                cb, sl = c
                coords = load_block(base)
                for kk in range(GSZ):
                    bp = bvec[kk]

                    def do_flush(cs):
                        flush(cs[0], cs[1])
                        return (bp, cs[1] + 1)

                    cb, sl = lax.cond(bp != cb, do_flush, lambda cs: cs, (cb, sl))
                    process(coords, kk)
                return (cb, sl)

            return lax.cond(same, fast, slow, carry)

        b0 = bb[pl.ds(0, L)][0]
        cur_b, slot = lax.fori_loop(
            0, NGROUPS, group_body, (b0, jnp.int32(0))
        )
        flush(cur_b, slot)
        pltpu.sync_copy(idsb, ids_hbm.at[wid])

    return k(x0, x1, x2, batch, v_flat)


def _tc_reduce(parts, ids3, idcol):
    def body(ids_ref, idcol_ref, parts_ref, out_ref, acc_ref):
        kk = pl.program_id(0)
        ids_row = ids_ref[0]
        idc = idcol_ref[...]
        rows = jnp.where(idc[:, 0:1] >= 0, parts_ref[...], 0.0)
        bio = lax.broadcasted_iota(jnp.int32, (B, CH), 0)
        mask = (jnp.broadcast_to(ids_row, (B, CH)) == bio).astype(jnp.float32)
        contrib = jax.lax.dot(
            mask, rows,
            precision=lax.Precision.HIGHEST,
            preferred_element_type=jnp.float32,
        )

        @pl.when(kk == 0)
        def _():
            acc_ref[...] = contrib

        @pl.when(kk > 0)
        def _():
            acc_ref[...] = acc_ref[...] + contrib

        @pl.when(kk == NSTEPS - 1)
        def _():
            h = acc_ref[...]
            s = 1
            while s < RES:
                h = h + jnp.concatenate(
                    [jnp.zeros((B, s * T), jnp.float32), h[:, : (RES - s) * T]],
                    axis=1,
                )
                s *= 2
            out_ref[...] = h

    return pl.pallas_call(
        body,
        grid=(NSTEPS,),
        in_specs=[
            pl.BlockSpec((1, 1, CH), lambda k: (k, 0, 0)),
            pl.BlockSpec((CH, 128), lambda k: (k, 0)),
            pl.BlockSpec((CH, HSZ), lambda k: (k, 0)),
        ],
        out_specs=pl.BlockSpec((B, HSZ), lambda k: (0, 0)),
        out_shape=jax.ShapeDtypeStruct((B, HSZ), jnp.float32),
        scratch_shapes=[pltpu.VMEM((B, HSZ), jnp.float32)],
    )(ids3, idcol, parts)


def kernel(x, batch, v):
    xt = x.T
    b32 = batch.astype(jnp.int32)
    vf = v.reshape(-1)
    parts, ids = _sc_hist(xt[0], xt[1], xt[2], b32, vf)
    ids_flat = ids.reshape(SLOTS)
    flat = _tc_reduce(
        parts.reshape(SLOTS, HSZ),
        ids_flat.reshape(NSTEPS, 1, CH),
        jnp.broadcast_to(ids_flat[:, None], (SLOTS, 128)),
    )
    return flat.reshape(B, RES, T)
